# baseline (speedup 1.0000x reference)
import functools

import jax
import jax.numpy as jnp
from jax import lax
from jax.experimental import pallas as pl
from jax.experimental.pallas import tpu as pltpu
from jax.experimental.pallas import tpu_sc as plsc

N = 10000
D = 128
E = 320000
CHUNK = 128
NC, NS = 2, 16
NW = NC * NS
NCH = 80
EPT = NCH * CHUNK
EP = NW * EPT
ROWS_PER_TILE = 624
TAIL_ROWS = N - NS * ROWS_PER_TILE
ZROWS = 16
ALPHA_PAD = 224


def _sc_aggregate(x, src, dst, etype, alpha_sym):
    mesh = plsc.VectorSubcoreMesh(core_axis_name="c", subcore_axis_name="s")

    @functools.partial(
        pl.kernel,
        mesh=mesh,
        out_type=jax.ShapeDtypeStruct((NC, N, D), jnp.float32),
        scratch_types=[
            pltpu.VMEM((EPT,), jnp.int32),
            pltpu.VMEM((CHUNK,), jnp.int32),
            pltpu.VMEM((CHUNK,), jnp.int32),
            pltpu.VMEM((CHUNK,), jnp.int32),
            pltpu.VMEM((CHUNK,), jnp.int32),
            pltpu.VMEM((CHUNK, D), jnp.float32),
            pltpu.VMEM((CHUNK, D), jnp.float32),
            pltpu.VMEM((ALPHA_PAD,), jnp.float32),
            pltpu.VMEM((ZROWS, D), jnp.float32),
            pltpu.VMEM_SHARED((N, D), jnp.float32),
            pltpu.SemaphoreType.DMA,
            pltpu.SemaphoreType.DMA,
        ],
    )
    def k(x_hbm, src_hbm, dst_hbm, type_hbm, alpha_hbm, out_hbm,
          srcs_v, type0, type1, dst0, dst1, rows0, rows1, alpha_v, zero_v,
          acc_sh, gsem, dsem):
        cid = lax.axis_index("c")
        sid = lax.axis_index("s")
        wid = sid * NC + cid

        pltpu.sync_copy(alpha_hbm, alpha_v)
        ebase = wid * EPT
        pltpu.sync_copy(src_hbm.at[pl.ds(ebase, EPT)], srcs_v)

        z16 = jnp.zeros((16,), jnp.float32)

        def zfill(i, _):
            zero_v[i // 8, pl.ds((i % 8) * 16, 16)] = z16
            return 0

        lax.fori_loop(0, ZROWS * 8, zfill, 0)
        base = sid * ROWS_PER_TILE

        def zcopy(i, _):
            pltpu.sync_copy(zero_v, acc_sh.at[pl.ds(base + i * ZROWS, ZROWS)])
            return 0

        lax.fori_loop(0, ROWS_PER_TILE // ZROWS, zcopy, 0)

        @pl.when(sid == NS - 1)
        def _zero_tail():
            pltpu.sync_copy(zero_v.at[pl.ds(0, TAIL_ROWS)],
                            acc_sh.at[pl.ds(NS * ROWS_PER_TILE, TAIL_ROWS)])

        plsc.subcore_barrier()

        def g_start(c, buf, dbuf, tbuf):
            pltpu.async_copy(x_hbm.at[srcs_v.at[pl.ds(c * CHUNK, CHUNK)]],
                             buf, gsem)
            pltpu.async_copy(dst_hbm.at[pl.ds(ebase + c * CHUNK, CHUNK)],
                             dbuf, dsem)
            pltpu.async_copy(type_hbm.at[pl.ds(ebase + c * CHUNK, CHUNK)],
                             tbuf, dsem)

        def g_wait(buf, dbuf, tbuf):
            pltpu.make_async_copy(
                x_hbm.at[srcs_v.at[pl.ds(0, CHUNK)]], buf, gsem).wait()
            pltpu.make_async_copy(
                dst_hbm.at[pl.ds(0, CHUNK)], dbuf, dsem).wait()
            pltpu.make_async_copy(
                type_hbm.at[pl.ds(0, CHUNK)], tbuf, dsem).wait()

        def scale(buf, tbuf):
            def grp(g, _):
                tv = tbuf[pl.ds(g * 16, 16)]
                for k16 in range(16):
                    e = g * 16 + k16
                    av = alpha_v[pl.ds(tv[k16], 16)]
                    a_spl = jnp.full((16,), av[0], jnp.float32)
                    for cg in range(8):
                        sl = pl.ds(cg * 16, 16)
                        buf[e, sl] = buf[e, sl] * a_spl
                return 0

            lax.fori_loop(0, CHUNK // 16, grp, 0)

        def scat(buf, dbuf):
            pltpu.sync_copy(buf, acc_sh.at[dbuf], add=True)

        g_start(0, rows0, dst0, type0)

        def pair(i, _):
            c0 = 2 * i
            g_wait(rows0, dst0, type0)
            g_start(c0 + 1, rows1, dst1, type1)
            scale(rows0, type0)
            scat(rows0, dst0)
            g_wait(rows1, dst1, type1)
            g_start(c0 + 2, rows0, dst0, type0)
            scale(rows1, type1)
            scat(rows1, dst1)
            return 0

        lax.fori_loop(0, (NCH - 2) // 2, pair, 0)
        g_wait(rows0, dst0, type0)
        g_start(NCH - 1, rows1, dst1, type1)
        scale(rows0, type0)
        scat(rows0, dst0)
        g_wait(rows1, dst1, type1)
        scale(rows1, type1)
        scat(rows1, dst1)

        plsc.subcore_barrier()

        pltpu.sync_copy(acc_sh.at[pl.ds(base, ROWS_PER_TILE)],
                        out_hbm.at[cid, pl.ds(base, ROWS_PER_TILE)])

        @pl.when(sid == NS - 1)
        def _drain_tail():
            pltpu.sync_copy(acc_sh.at[pl.ds(NS * ROWS_PER_TILE, TAIL_ROWS)],
                            out_hbm.at[cid, pl.ds(NS * ROWS_PER_TILE, TAIL_ROWS)])

    return k(x, src, dst, etype, alpha_sym)


def _tc_finish(partials, W, gamma, beta):
    def body(p_ref, w_ref, g_ref, b_ref, o_ref):
        agg = p_ref[0] + p_ref[1]
        feats = jnp.dot(agg, w_ref[...], preferred_element_type=jnp.float32)
        mean = jnp.mean(feats, axis=0, keepdims=True)
        dd = feats - mean
        var = jnp.mean(dd * dd, axis=0, keepdims=True)
        o_ref[...] = dd * lax.rsqrt(var + 1e-5) * g_ref[...] + b_ref[...]

    return pl.pallas_call(
        body,
        out_shape=jax.ShapeDtypeStruct((N, D), jnp.float32),
    )(partials, W, gamma.reshape(1, D), beta.reshape(1, D))


def kernel(x, edge_index, all_edge_type, W, alpha_table, gamma, beta):
    num_rel = alpha_table.shape[0]
    half = num_rel // 2
    table = alpha_table.at[0].set(0.0)[:, 0]
    r = jnp.arange(num_rel)
    transposed = jnp.where(r >= half, r - half, r + half)
    alpha_sym = jnp.pad(table + table[transposed], (0, ALPHA_PAD - num_rel))

    pad = EP - E
    srcp = jnp.concatenate([edge_index[0], jnp.zeros((pad,), jnp.int32)])
    dstp = jnp.concatenate([edge_index[1], jnp.zeros((pad,), jnp.int32)])
    typep = jnp.concatenate([all_edge_type, jnp.full((pad,), num_rel, jnp.int32)])

    partials = _sc_aggregate(x, srcp, dstp, typep, alpha_sym)
    return _tc_finish(partials, W, gamma, beta)


# speedup vs baseline: 2.1614x; 2.1614x over previous
import functools

import jax
import jax.numpy as jnp
from jax import lax
from jax.experimental import pallas as pl
from jax.experimental.pallas import tpu as pltpu
from jax.experimental.pallas import tpu_sc as plsc

N = 10000
D = 128
E = 320000
CHUNK = 128
NC, NS = 2, 16
NW = NC * NS
NCH = 80
EPT = NCH * CHUNK
EP = NW * EPT
ROWS_PER_TILE = 624
TAIL_ROWS = N - NS * ROWS_PER_TILE
ZROWS = 16
ALPHA_PAD = 224


def _sc_aggregate(x, src, dst, etype, alpha_sym):
    mesh = plsc.VectorSubcoreMesh(core_axis_name="c", subcore_axis_name="s")

    @functools.partial(
        pl.kernel,
        mesh=mesh,
        out_type=jax.ShapeDtypeStruct((NC, N, D), jnp.float32),
        scratch_types=[
            pltpu.VMEM((EPT,), jnp.int32),
            pltpu.VMEM((CHUNK,), jnp.int32),
            pltpu.VMEM((CHUNK,), jnp.int32),
            pltpu.VMEM((CHUNK,), jnp.int32),
            pltpu.VMEM((CHUNK,), jnp.int32),
            pltpu.VMEM((CHUNK, D), jnp.float32),
            pltpu.VMEM((CHUNK, D), jnp.float32),
            pltpu.VMEM((ALPHA_PAD,), jnp.float32),
            pltpu.VMEM((ZROWS, D), jnp.float32),
            pltpu.VMEM_SHARED((N, D), jnp.float32),
            pltpu.SemaphoreType.DMA,
            pltpu.SemaphoreType.DMA,
        ],
    )
    def k(x_hbm, src_hbm, dst_hbm, type_hbm, alpha_hbm, out_hbm,
          srcs_v, type0, type1, dst0, dst1, rows0, rows1, alpha_v, zero_v,
          acc_sh, gsem, dsem):
        cid = lax.axis_index("c")
        sid = lax.axis_index("s")
        wid = sid * NC + cid

        pltpu.sync_copy(alpha_hbm, alpha_v)
        ebase = wid * EPT
        pltpu.sync_copy(src_hbm.at[pl.ds(ebase, EPT)], srcs_v)

        z16 = jnp.zeros((16,), jnp.float32)

        def zfill(i, _):
            zero_v[i // 8, pl.ds((i % 8) * 16, 16)] = z16
            return 0

        lax.fori_loop(0, ZROWS * 8, zfill, 0)
        base = sid * ROWS_PER_TILE

        def zcopy(i, _):
            pltpu.sync_copy(zero_v, acc_sh.at[pl.ds(base + i * ZROWS, ZROWS)])
            return 0

        lax.fori_loop(0, ROWS_PER_TILE // ZROWS, zcopy, 0)

        @pl.when(sid == NS - 1)
        def _zero_tail():
            pltpu.sync_copy(zero_v.at[pl.ds(0, TAIL_ROWS)],
                            acc_sh.at[pl.ds(NS * ROWS_PER_TILE, TAIL_ROWS)])

        plsc.subcore_barrier()

        def g_start(c, buf, dbuf, tbuf):
            pltpu.async_copy(x_hbm.at[srcs_v.at[pl.ds(c * CHUNK, CHUNK)]],
                             buf, gsem)
            pltpu.async_copy(dst_hbm.at[pl.ds(ebase + c * CHUNK, CHUNK)],
                             dbuf, dsem)
            pltpu.async_copy(type_hbm.at[pl.ds(ebase + c * CHUNK, CHUNK)],
                             tbuf, dsem)

        def g_wait(buf, dbuf, tbuf):
            pltpu.make_async_copy(
                x_hbm.at[srcs_v.at[pl.ds(0, CHUNK)]], buf, gsem).wait()
            pltpu.make_async_copy(
                dst_hbm.at[pl.ds(0, CHUNK)], dbuf, dsem).wait()
            pltpu.make_async_copy(
                type_hbm.at[pl.ds(0, CHUNK)], tbuf, dsem).wait()

        def scale(buf, tbuf):
            def grp(g, _):
                tv = tbuf[pl.ds(g * 16, 16)]
                for k16 in range(16):
                    e = g * 16 + k16
                    av = alpha_v[pl.ds(tv[k16], 16)]
                    a_spl = jnp.full((16,), av[0], jnp.float32)
                    for cg in range(8):
                        sl = pl.ds(cg * 16, 16)
                        buf[e, sl] = buf[e, sl] * a_spl
                return 0

            lax.fori_loop(0, CHUNK // 16, grp, 0)

        def scat(buf, dbuf):
            pltpu.sync_copy(buf, acc_sh.at[dbuf], add=True)

        g_start(0, rows0, dst0, type0)

        def pair(i, _):
            c0 = 2 * i
            g_wait(rows0, dst0, type0)
            g_start(c0 + 1, rows1, dst1, type1)
            scale(rows0, type0)
            scat(rows0, dst0)
            g_wait(rows1, dst1, type1)
            g_start(c0 + 2, rows0, dst0, type0)
            scale(rows1, type1)
            scat(rows1, dst1)
            return 0

        lax.fori_loop(0, (NCH - 2) // 2, pair, 0)
        g_wait(rows0, dst0, type0)
        g_start(NCH - 1, rows1, dst1, type1)
        scale(rows0, type0)
        scat(rows0, dst0)
        g_wait(rows1, dst1, type1)
        scale(rows1, type1)
        scat(rows1, dst1)

        plsc.subcore_barrier()

        pltpu.sync_copy(acc_sh.at[pl.ds(base, ROWS_PER_TILE)],
                        out_hbm.at[cid, pl.ds(base, ROWS_PER_TILE)])

        @pl.when(sid == NS - 1)
        def _drain_tail():
            pltpu.sync_copy(acc_sh.at[pl.ds(NS * ROWS_PER_TILE, TAIL_ROWS)],
                            out_hbm.at[cid, pl.ds(NS * ROWS_PER_TILE, TAIL_ROWS)])

    return k(x, src, dst, etype, alpha_sym)


def _tc_finish(partials, W, gamma, beta):
    def body(p_ref, w_ref, g_ref, b_ref, o_ref):
        agg = p_ref[0] + p_ref[1]
        feats = jnp.dot(agg, w_ref[...], preferred_element_type=jnp.float32)
        mean = jnp.mean(feats, axis=0, keepdims=True)
        dd = feats - mean
        var = jnp.mean(dd * dd, axis=0, keepdims=True)
        o_ref[...] = dd * lax.rsqrt(var + 1e-5) * g_ref[...] + b_ref[...]

    return pl.pallas_call(
        body,
        out_shape=jax.ShapeDtypeStruct((N, D), jnp.float32),
    )(partials, W, gamma.reshape(1, D), beta.reshape(1, D))


def kernel(x, edge_index, all_edge_type, W, alpha_table, gamma, beta):
    num_rel = alpha_table.shape[0]
    half = num_rel // 2
    table = alpha_table.at[0].set(0.0)[:, 0]
    r = jnp.arange(num_rel)
    transposed = jnp.where(r >= half, r - half, r + half)
    alpha_sym = jnp.pad(table + table[transposed], (0, ALPHA_PAD - num_rel))

    pad = EP - E
    spread = jnp.arange(pad, dtype=jnp.int32) % N
    srcp = jnp.concatenate([edge_index[0], spread])
    dstp = jnp.concatenate([edge_index[1], spread])
    typep = jnp.concatenate([all_edge_type, jnp.full((pad,), num_rel, jnp.int32)])

    partials = _sc_aggregate(x, srcp, dstp, typep, alpha_sym)
    return _tc_finish(partials, W, gamma, beta)


# speedup vs baseline: 2.3354x; 1.0805x over previous
import functools

import jax
import jax.numpy as jnp
from jax import lax
from jax.experimental import pallas as pl
from jax.experimental.pallas import tpu as pltpu
from jax.experimental.pallas import tpu_sc as plsc

N = 10000
D = 128
E = 320000
CHUNK = 96
NC, NS = 2, 16
NW = NC * NS
NCH = 105
EPT = NCH * CHUNK
EP = NW * EPT
ROWS_PER_TILE = 624
TAIL_ROWS = N - NS * ROWS_PER_TILE
ZROWS = 16
ALPHA_PAD = 224


def _sc_aggregate(x, src, dst, etype, alpha_sym):
    mesh = plsc.VectorSubcoreMesh(core_axis_name="c", subcore_axis_name="s")

    @functools.partial(
        pl.kernel,
        mesh=mesh,
        out_type=jax.ShapeDtypeStruct((NC, N, D), jnp.float32),
        scratch_types=[
            pltpu.VMEM((EPT,), jnp.int32),
            pltpu.VMEM((CHUNK,), jnp.int32),
            pltpu.VMEM((CHUNK,), jnp.int32),
            pltpu.VMEM((CHUNK,), jnp.int32),
            pltpu.VMEM((CHUNK,), jnp.int32),
            pltpu.VMEM((CHUNK,), jnp.int32),
            pltpu.VMEM((CHUNK,), jnp.int32),
            pltpu.VMEM((CHUNK, D), jnp.float32),
            pltpu.VMEM((CHUNK, D), jnp.float32),
            pltpu.VMEM((CHUNK, D), jnp.float32),
            pltpu.VMEM((ALPHA_PAD,), jnp.float32),
            pltpu.VMEM((ZROWS, D), jnp.float32),
            pltpu.VMEM_SHARED((N, D), jnp.float32),
            pltpu.SemaphoreType.DMA,
            pltpu.SemaphoreType.DMA,
            pltpu.SemaphoreType.DMA,
            pltpu.SemaphoreType.DMA,
            pltpu.SemaphoreType.DMA,
        ],
    )
    def k(x_hbm, src_hbm, dst_hbm, type_hbm, alpha_hbm, out_hbm,
          srcs_v, type0, type1, type2, dst0, dst1, dst2, rows0, rows1, rows2,
          alpha_v, zero_v, acc_sh, gsem, dsem, ssem0, ssem1, ssem2):
        cid = lax.axis_index("c")
        sid = lax.axis_index("s")
        wid = sid * NC + cid

        buffers = ((rows0, dst0, type0, ssem0),
                   (rows1, dst1, type1, ssem1),
                   (rows2, dst2, type2, ssem2))

        pltpu.sync_copy(alpha_hbm, alpha_v)
        ebase = wid * EPT
        pltpu.sync_copy(src_hbm.at[pl.ds(ebase, EPT)], srcs_v)

        def g_start(c, b):
            rows_b, dst_b, type_b, _ = buffers[b]
            pltpu.async_copy(x_hbm.at[srcs_v.at[pl.ds(c * CHUNK, CHUNK)]],
                             rows_b, gsem)
            pltpu.async_copy(dst_hbm.at[pl.ds(ebase + c * CHUNK, CHUNK)],
                             dst_b, dsem)
            pltpu.async_copy(type_hbm.at[pl.ds(ebase + c * CHUNK, CHUNK)],
                             type_b, dsem)

        def g_wait(b):
            rows_b, dst_b, type_b, _ = buffers[b]
            pltpu.make_async_copy(
                x_hbm.at[srcs_v.at[pl.ds(0, CHUNK)]], rows_b, gsem).wait()
            pltpu.make_async_copy(
                dst_hbm.at[pl.ds(0, CHUNK)], dst_b, dsem).wait()
            pltpu.make_async_copy(
                type_hbm.at[pl.ds(0, CHUNK)], type_b, dsem).wait()

        def s_start(b):
            rows_b, dst_b, _, ssem_b = buffers[b]
            pltpu.async_copy(rows_b, acc_sh.at[dst_b], ssem_b, add=True)

        def s_wait(b):
            rows_b, dst_b, _, ssem_b = buffers[b]
            pltpu.make_async_copy(rows_b, acc_sh.at[dst_b], ssem_b).wait()

        def scale(b):
            rows_b, _, type_b, _ = buffers[b]

            def grp(g, _):
                tv = type_b[pl.ds(g * 16, 16)]
                for k16 in range(16):
                    e = g * 16 + k16
                    av = alpha_v[pl.ds(tv[k16], 16)]
                    a_spl = jnp.full((16,), av[0], jnp.float32)
                    for cg in range(8):
                        sl = pl.ds(cg * 16, 16)
                        rows_b[e, sl] = rows_b[e, sl] * a_spl
                return 0

            lax.fori_loop(0, CHUNK // 16, grp, 0)

        g_start(0, 0)

        z16 = jnp.zeros((16,), jnp.float32)

        def zfill(i, _):
            zero_v[i // 8, pl.ds((i % 8) * 16, 16)] = z16
            return 0

        lax.fori_loop(0, ZROWS * 8, zfill, 0)
        base = sid * ROWS_PER_TILE

        def zcopy(i, _):
            pltpu.sync_copy(zero_v, acc_sh.at[pl.ds(base + i * ZROWS, ZROWS)])
            return 0

        lax.fori_loop(0, ROWS_PER_TILE // ZROWS, zcopy, 0)

        @pl.when(sid == NS - 1)
        def _zero_tail():
            pltpu.sync_copy(zero_v.at[pl.ds(0, TAIL_ROWS)],
                            acc_sh.at[pl.ds(NS * ROWS_PER_TILE, TAIL_ROWS)])

        plsc.subcore_barrier()

        def step(c, bcur, bnext, first, last):
            g_wait(bcur)
            if not last:
                if not first:
                    s_wait(bnext)
                g_start(c + 1, bnext)
            scale(bcur)
            s_start(bcur)

        step(0, 0, 1, True, False)
        step(1, 1, 2, True, False)

        def triple(i, _):
            c = 2 + 3 * i
            step(c, 2, 0, False, False)
            step(c + 1, 0, 1, False, False)
            step(c + 2, 1, 2, False, False)
            return 0

        lax.fori_loop(0, (NCH - 3) // 3, triple, 0)
        step(NCH - 1, 2, 0, False, True)

        s_wait(0)
        s_wait(1)
        s_wait(2)
        plsc.subcore_barrier()

        pltpu.sync_copy(acc_sh.at[pl.ds(base, ROWS_PER_TILE)],
                        out_hbm.at[cid, pl.ds(base, ROWS_PER_TILE)])

        @pl.when(sid == NS - 1)
        def _drain_tail():
            pltpu.sync_copy(acc_sh.at[pl.ds(NS * ROWS_PER_TILE, TAIL_ROWS)],
                            out_hbm.at[cid, pl.ds(NS * ROWS_PER_TILE, TAIL_ROWS)])

    return k(x, src, dst, etype, alpha_sym)


def _tc_finish(partials, W, gamma, beta):
    def body(p_ref, w_ref, g_ref, b_ref, o_ref):
        agg = p_ref[0] + p_ref[1]
        feats = jnp.dot(agg, w_ref[...], preferred_element_type=jnp.float32)
        mean = jnp.mean(feats, axis=0, keepdims=True)
        dd = feats - mean
        var = jnp.mean(dd * dd, axis=0, keepdims=True)
        o_ref[...] = dd * lax.rsqrt(var + 1e-5) * g_ref[...] + b_ref[...]

    return pl.pallas_call(
        body,
        out_shape=jax.ShapeDtypeStruct((N, D), jnp.float32),
    )(partials, W, gamma.reshape(1, D), beta.reshape(1, D))


def kernel(x, edge_index, all_edge_type, W, alpha_table, gamma, beta):
    num_rel = alpha_table.shape[0]
    half = num_rel // 2
    table = alpha_table.at[0].set(0.0)[:, 0]
    r = jnp.arange(num_rel)
    transposed = jnp.where(r >= half, r - half, r + half)
    alpha_sym = jnp.pad(table + table[transposed], (0, ALPHA_PAD - num_rel))

    pad = EP - E
    spread = jnp.arange(pad, dtype=jnp.int32) % N
    srcp = jnp.concatenate([edge_index[0], spread])
    dstp = jnp.concatenate([edge_index[1], spread])
    typep = jnp.concatenate([all_edge_type, jnp.full((pad,), num_rel, jnp.int32)])

    partials = _sc_aggregate(x, srcp, dstp, typep, alpha_sym)
    return _tc_finish(partials, W, gamma, beta)
